# baseline (speedup 1.0000x reference)
import functools

import jax
import jax.numpy as jnp
from jax import lax
from jax.experimental import pallas as pl
from jax.experimental.pallas import tpu as pltpu
from jax.experimental.pallas import tpu_sc as plsc

_B = 16384
_D = 32


def _sc_gather(user, movie, gmf_user, gmf_movie, mlp_user, mlp_movie):
    info = plsc.get_sparse_core_info()
    nw = info.num_cores * info.num_subcores
    bpw = _B // nw
    mesh = plsc.VectorSubcoreMesh(core_axis_name="c", subcore_axis_name="s")

    @functools.partial(
        pl.kernel,
        mesh=mesh,
        compiler_params=pltpu.CompilerParams(use_tc_tiling_on_sc=False),
        out_type=[jax.ShapeDtypeStruct((_B, _D), jnp.float32)] * 4,
        scratch_types=[
            pltpu.VMEM((bpw,), jnp.int32),
            pltpu.VMEM((bpw,), jnp.int32),
            pltpu.VMEM((bpw, _D), jnp.float32),
            pltpu.VMEM((bpw, _D), jnp.float32),
            pltpu.VMEM((bpw, _D), jnp.float32),
            pltpu.VMEM((bpw, _D), jnp.float32),
            pltpu.SemaphoreType.DMA,
        ],
    )
    def k(user_h, movie_h, gu_h, gm_h, mu_h, mm_h,
          gu_o, gm_o, mu_o, mm_o,
          uidx, midx, gu_v, gm_v, mu_v, mm_v, sem):
        wid = lax.axis_index("s") * info.num_cores + lax.axis_index("c")
        base = wid * bpw
        pltpu.sync_copy(user_h.at[pl.ds(base, bpw)], uidx)
        pltpu.sync_copy(movie_h.at[pl.ds(base, bpw)], midx)
        c1 = pltpu.async_copy(gu_h.at[uidx], gu_v, sem)
        c2 = pltpu.async_copy(gm_h.at[midx], gm_v, sem)
        c3 = pltpu.async_copy(mu_h.at[uidx], mu_v, sem)
        c4 = pltpu.async_copy(mm_h.at[midx], mm_v, sem)
        c1.wait()
        c2.wait()
        c3.wait()
        c4.wait()
        pltpu.sync_copy(gu_v, gu_o.at[pl.ds(base, bpw)])
        pltpu.sync_copy(gm_v, gm_o.at[pl.ds(base, bpw)])
        pltpu.sync_copy(mu_v, mu_o.at[pl.ds(base, bpw)])
        pltpu.sync_copy(mm_v, mm_o.at[pl.ds(base, bpw)])

    return k(user, movie, gmf_user, gmf_movie, mlp_user, mlp_movie)


def _mlp_body(gu_r, gm_r, mu_r, mm_r, w1_r, b1_r, w2_r, b2_r, w3_r, b3_r,
              wo_r, bo_r, out_r):
    f32 = jnp.float32
    w1 = w1_r[...]
    h = (lax.dot_general(mu_r[...], w1[:, :_D], (((1,), (1,)), ((), ())),
                         preferred_element_type=f32)
         + lax.dot_general(mm_r[...], w1[:, _D:], (((1,), (1,)), ((), ())),
                           preferred_element_type=f32)
         + b1_r[...])
    h = jnp.maximum(h, 0.0)
    h = lax.dot_general(h, w2_r[...], (((1,), (1,)), ((), ())),
                        preferred_element_type=f32) + b2_r[...]
    h = jnp.maximum(h, 0.0)
    h = lax.dot_general(h, w3_r[...], (((1,), (1,)), ((), ())),
                        preferred_element_type=f32) + b3_r[...]
    h = jnp.maximum(h, 0.0)
    gmf = gu_r[...] * gm_r[...]
    wo = wo_r[...]
    out = (jnp.sum(gmf * wo[:, :_D], axis=1)
           + jnp.sum(h * wo[:, _D:], axis=1)
           + bo_r[0, 0])
    out_r[...] = out


def _tc_mlp(gu, gm, mu, mm, W1, b1, W2, b2, W3, b3, Wo, bo):
    blk = 2048
    grid = _B // blk
    row_spec = pl.BlockSpec((blk, _D), lambda i: (i, 0))

    def full(shape):
        return pl.BlockSpec(shape, lambda i: (0,) * len(shape))

    return pl.pallas_call(
        _mlp_body,
        grid=(grid,),
        in_specs=[
            row_spec, row_spec, row_spec, row_spec,
            full(W1.shape), full((1, 128)),
            full(W2.shape), full((1, 64)),
            full(W3.shape), full((1, 32)),
            full(Wo.shape), full((1, 1)),
        ],
        out_specs=pl.BlockSpec((blk,), lambda i: (i,)),
        out_shape=jax.ShapeDtypeStruct((_B,), jnp.float32),
    )(gu, gm, mu, mm, W1, b1.reshape(1, 128), W2, b2.reshape(1, 64),
      W3, b3.reshape(1, 32), Wo, bo.reshape(1, 1))


def kernel(user, movie, gmf_user, gmf_movie, mlp_user, mlp_movie,
           W1, b1, W2, b2, W3, b3, Wo, bo):
    user = user.astype(jnp.int32)
    movie = movie.astype(jnp.int32)
    gu, gm, mu, mm = _sc_gather(user, movie, gmf_user, gmf_movie,
                                mlp_user, mlp_movie)
    return _tc_mlp(gu, gm, mu, mm, W1, b1, W2, b2, W3, b3, Wo, bo)
